# baseline (speedup 1.0000x reference)
import functools

import jax
import jax.numpy as jnp
from jax import lax
from jax.experimental import pallas as pl
from jax.experimental.pallas import tpu as pltpu
from jax.experimental.pallas import tpu_sc as plsc

TOP_K = 2
N_GROUP = 4
TOPK_GROUP = 2
ROUTED_SCALING = 2.5
NEG = -1e30
BLK = 128


def _relu2(x):
    r = jnp.maximum(x, 0.0)
    return r * r


def _gate_topk(x_blk, gw, gb):
    bT = x_blk.shape[0]
    E = gw.shape[0]
    gsz = E // N_GROUP
    logits = lax.dot_general(
        x_blk, gw, (((1,), (1,)), ((), ())),
        preferred_element_type=jnp.float32)
    scores = jax.nn.sigmoid(logits)
    sfc = scores + gb

    def top2sum4(v):
        a, b = v[:, 0:1], v[:, 1:2]
        c, d = v[:, 2:3], v[:, 3:4]
        m_ab, n_ab = jnp.maximum(a, b), jnp.minimum(a, b)
        m_cd, n_cd = jnp.maximum(c, d), jnp.minimum(c, d)
        top1 = jnp.maximum(m_ab, m_cd)
        top2 = jnp.maximum(jnp.minimum(m_ab, m_cd), jnp.maximum(n_ab, n_cd))
        return top1 + top2

    gs = [top2sum4(sfc[:, g * gsz:(g + 1) * gsz]) for g in range(N_GROUP)]
    m_ab, n_ab = jnp.maximum(gs[0], gs[1]), jnp.minimum(gs[0], gs[1])
    m_cd, n_cd = jnp.maximum(gs[2], gs[3]), jnp.minimum(gs[2], gs[3])
    thresh = jnp.maximum(jnp.minimum(m_ab, m_cd), jnp.maximum(n_ab, n_cd))

    lane = lax.broadcasted_iota(jnp.int32, (bT, E), 1)
    gid = lane // gsz
    emask = jnp.zeros((bT, E), jnp.float32)
    for g in range(N_GROUP):
        emask = emask + jnp.where(gid == g, 1.0, 0.0) * (gs[g] >= thresh)
    masked = jnp.where(emask > 0, sfc, 0.0)

    v1 = jnp.max(masked, axis=1, keepdims=True)
    idx1 = jnp.min(jnp.where(masked == v1, lane, E), axis=1, keepdims=True)
    sel1 = (lane == idx1)
    masked2 = jnp.where(sel1, NEG, masked)
    v2 = jnp.max(masked2, axis=1, keepdims=True)
    idx2 = jnp.min(jnp.where(masked2 == v2, lane, E), axis=1, keepdims=True)
    sel2 = (lane == idx2)

    w1v = jnp.sum(jnp.where(sel1, scores, 0.0), axis=1, keepdims=True)
    w2v = jnp.sum(jnp.where(sel2, scores, 0.0), axis=1, keepdims=True)
    denom = w1v + w2v + 1e-20
    return sel1, sel2, ROUTED_SCALING * w1v / denom, ROUTED_SCALING * w2v / denom


def _routing_kernel(x_ref, gw_ref, gb_ref, eidx_ref, rnk_ref, wts_ref,
                    cnt_ref, run_ref):
    t = pl.program_id(0)
    bT = x_ref.shape[0]
    E = gw_ref.shape[0]

    @pl.when(t == 0)
    def _():
        run_ref[...] = jnp.zeros_like(run_ref)

    sel1, sel2, w1n, w2n = _gate_topk(x_ref[...], gw_ref[...], gb_ref[...])
    lane = lax.broadcasted_iota(jnp.int32, (bT, E), 1)
    idx1 = jnp.sum(jnp.where(sel1, lane, 0), axis=1, keepdims=True)
    idx2 = jnp.sum(jnp.where(sel2, lane, 0), axis=1, keepdims=True)

    oh = sel1.astype(jnp.float32) + sel2.astype(jnp.float32)
    row = lax.broadcasted_iota(jnp.int32, (bT, bT), 0)
    col = lax.broadcasted_iota(jnp.int32, (bT, bT), 1)
    ltri = jnp.where(row > col, 1.0, 0.0)
    prefix = lax.dot_general(
        ltri, oh, (((1,), (0,)), ((), ())),
        preferred_element_type=jnp.float32)
    base = run_ref[...] + prefix
    r1 = jnp.sum(jnp.where(sel1, base, 0.0), axis=1, keepdims=True)
    r2 = jnp.sum(jnp.where(sel2, base, 0.0), axis=1, keepdims=True)
    run_new = run_ref[...] + jnp.sum(oh, axis=0, keepdims=True)
    run_ref[...] = run_new

    eidx_ref[...] = jnp.concatenate([idx1, idx2], axis=1)
    rnk_ref[...] = jnp.concatenate([r1, r2], axis=1).astype(jnp.int32)
    wts_ref[...] = jnp.concatenate([w1n, w2n], axis=1)
    cnt_ref[...] = run_new.astype(jnp.int32)


def _grouped_mlp_kernel(be_ref, xs_ref, w1_ref, w2_ref, ys_ref):
    del be_ref
    h = _relu2(lax.dot_general(
        xs_ref[...].astype(jnp.bfloat16), w1_ref[0], (((1,), (1,)), ((), ())),
        preferred_element_type=jnp.float32))
    ys_ref[...] = lax.dot_general(
        h.astype(jnp.bfloat16), w2_ref[0], (((1,), (1,)), ((), ())),
        preferred_element_type=jnp.float32)


def _shared_mlp_kernel(x_ref, sw1_ref, sw2_ref, out_ref):
    s = _relu2(lax.dot_general(
        x_ref[...], sw1_ref[...], (((1,), (1,)), ((), ())),
        preferred_element_type=jnp.float32))
    out_ref[...] = lax.dot_general(
        s, sw2_ref[...], (((1,), (1,)), ((), ())),
        preferred_element_type=jnp.float32)


def _combine_kernel(sh_ref, yg_ref, wts_ref, out_ref):
    y0 = yg_ref[:, 0, :]
    y1 = yg_ref[:, 1, :]
    w0 = wts_ref[:, 0:1]
    w1 = wts_ref[:, 1:2]
    out_ref[...] = sh_ref[...] + w0 * y0 + w1 * y1


def _make_sc_row_gather(V, D, B, chunk):
    info = plsc.get_sparse_core_info()
    NC, NS = info.num_cores, info.num_subcores
    NW = NC * NS
    assert B % (8 * NW) == 0 and D % 16 == 0
    b_per_w = B // NW
    assert b_per_w % chunk == 0 and chunk <= 128 and chunk % 8 == 0
    n_chunks = b_per_w // chunk
    mesh = plsc.VectorSubcoreMesh(core_axis_name="c", subcore_axis_name="s")

    @functools.partial(
        pl.kernel, mesh=mesh,
        out_type=jax.ShapeDtypeStruct((B, D), jnp.float32),
        scratch_types=[
            pltpu.VMEM((chunk,), jnp.int32),
            pltpu.VMEM((chunk, D), jnp.float32),
            pltpu.SemaphoreType.DMA,
        ],
    )
    def gather_k(table_hbm, idx_hbm, out_hbm, idx_v, rows_v, sem):
        wid = lax.axis_index("s") * NC + lax.axis_index("c")
        base = wid * b_per_w
        for c in range(n_chunks):
            cbase = base + c * chunk
            pltpu.sync_copy(idx_hbm.at[pl.ds(cbase, chunk)], idx_v)
            pltpu.async_copy(table_hbm.at[idx_v], rows_v, sem).wait()
            pltpu.sync_copy(rows_v, out_hbm.at[pl.ds(cbase, chunk)])

    return gather_k


@jax.jit
def kernel(hidden_states, gate_w, gate_bias, w1, w2, shared_w1, shared_w2):
    x = hidden_states
    T, D = x.shape
    E, I, _ = w1.shape
    SI = shared_w1.shape[0]
    NB = (T * TOP_K) // BLK + E
    S = NB * BLK

    bT = min(512, T)
    eidx, rnk, wts, counts = pl.pallas_call(
        _routing_kernel,
        grid=(T // bT,),
        in_specs=[
            pl.BlockSpec((bT, D), lambda t: (t, 0)),
            pl.BlockSpec((E, D), lambda t: (0, 0)),
            pl.BlockSpec((1, E), lambda t: (0, 0)),
        ],
        out_specs=[
            pl.BlockSpec((bT, TOP_K), lambda t: (t, 0)),
            pl.BlockSpec((bT, TOP_K), lambda t: (t, 0)),
            pl.BlockSpec((bT, TOP_K), lambda t: (t, 0)),
            pl.BlockSpec((1, E), lambda t: (0, 0)),
        ],
        out_shape=[
            jax.ShapeDtypeStruct((T, TOP_K), jnp.int32),
            jax.ShapeDtypeStruct((T, TOP_K), jnp.int32),
            jax.ShapeDtypeStruct((T, TOP_K), jnp.float32),
            jax.ShapeDtypeStruct((1, E), jnp.int32),
        ],
        scratch_shapes=[pltpu.VMEM((1, E), jnp.float32)],
    )(x, gate_w, gate_bias.reshape(1, E))

    c = counts[0]
    nb = (c + BLK - 1) // BLK
    block_expert = jnp.repeat(
        jnp.arange(E, dtype=jnp.int32), nb, total_repeat_length=NB)
    offsets = (jnp.cumsum(nb) - nb).astype(jnp.int32) * BLK
    pos = offsets[eidx] + rnk
    tok = jnp.broadcast_to(
        jnp.arange(T, dtype=jnp.int32)[:, None], (T, TOP_K))
    perm = (jnp.arange(S, dtype=jnp.int32) % T).at[pos.reshape(-1)].set(
        tok.reshape(-1))

    bT2 = min(512, T)
    shared_y = pl.pallas_call(
        _shared_mlp_kernel,
        grid=(T // bT2,),
        in_specs=[
            pl.BlockSpec((bT2, D), lambda t: (t, 0)),
            pl.BlockSpec((SI, D), lambda t: (0, 0)),
            pl.BlockSpec((D, SI), lambda t: (0, 0)),
        ],
        out_specs=pl.BlockSpec((bT2, D), lambda t: (t, 0)),
        out_shape=jax.ShapeDtypeStruct((T, D), jnp.float32),
    )(x, shared_w1, shared_w2)

    xs = _make_sc_row_gather(T, D, S, 96)(x, perm)

    ys = pl.pallas_call(
        _grouped_mlp_kernel,
        grid_spec=pltpu.PrefetchScalarGridSpec(
            num_scalar_prefetch=1,
            grid=(NB,),
            in_specs=[
                pl.BlockSpec((BLK, D), lambda b, be: (b, 0)),
                pl.BlockSpec((1, I, D), lambda b, be: (be[b], 0, 0)),
                pl.BlockSpec((1, D, I), lambda b, be: (be[b], 0, 0)),
            ],
            out_specs=pl.BlockSpec((BLK, D), lambda b, be: (b, 0)),
        ),
        out_shape=jax.ShapeDtypeStruct((S, D), jnp.float32),
    )(block_expert, xs, w1.astype(jnp.bfloat16), w2.astype(jnp.bfloat16))

    yg = _make_sc_row_gather(S, D, T * TOP_K, 64)(ys, pos.reshape(-1))
    yg = yg.reshape(T, TOP_K, D)

    out = pl.pallas_call(
        _combine_kernel,
        grid=(T // bT2,),
        in_specs=[
            pl.BlockSpec((bT2, D), lambda t: (t, 0)),
            pl.BlockSpec((bT2, TOP_K, D), lambda t: (t, 0, 0)),
            pl.BlockSpec((bT2, TOP_K), lambda t: (t, 0)),
        ],
        out_specs=pl.BlockSpec((bT2, D), lambda t: (t, 0)),
        out_shape=jax.ShapeDtypeStruct((T, D), jnp.float32),
    )(shared_y, yg, wts)
    return out


# speedup vs baseline: 1.3315x; 1.3315x over previous
import functools

import jax
import jax.numpy as jnp
from jax import lax
from jax.experimental import pallas as pl
from jax.experimental.pallas import tpu as pltpu
from jax.experimental.pallas import tpu_sc as plsc

TOP_K = 2
N_GROUP = 4
TOPK_GROUP = 2
ROUTED_SCALING = 2.5
NEG = -1e30
BLK = 512


def _relu2(x):
    r = jnp.maximum(x, 0.0)
    return r * r


def _gate_topk(x_blk, gw, gb):
    bT = x_blk.shape[0]
    E = gw.shape[0]
    gsz = E // N_GROUP
    logits = lax.dot_general(
        x_blk, gw, (((1,), (1,)), ((), ())),
        preferred_element_type=jnp.float32)
    scores = jax.nn.sigmoid(logits)
    sfc = scores + gb

    def top2sum4(v):
        a, b = v[:, 0:1], v[:, 1:2]
        c, d = v[:, 2:3], v[:, 3:4]
        m_ab, n_ab = jnp.maximum(a, b), jnp.minimum(a, b)
        m_cd, n_cd = jnp.maximum(c, d), jnp.minimum(c, d)
        top1 = jnp.maximum(m_ab, m_cd)
        top2 = jnp.maximum(jnp.minimum(m_ab, m_cd), jnp.maximum(n_ab, n_cd))
        return top1 + top2

    gs = [top2sum4(sfc[:, g * gsz:(g + 1) * gsz]) for g in range(N_GROUP)]
    m_ab, n_ab = jnp.maximum(gs[0], gs[1]), jnp.minimum(gs[0], gs[1])
    m_cd, n_cd = jnp.maximum(gs[2], gs[3]), jnp.minimum(gs[2], gs[3])
    thresh = jnp.maximum(jnp.minimum(m_ab, m_cd), jnp.maximum(n_ab, n_cd))

    lane = lax.broadcasted_iota(jnp.int32, (bT, E), 1)
    gid = lane // gsz
    emask = jnp.zeros((bT, E), jnp.float32)
    for g in range(N_GROUP):
        emask = emask + jnp.where(gid == g, 1.0, 0.0) * (gs[g] >= thresh)
    masked = jnp.where(emask > 0, sfc, 0.0)

    v1 = jnp.max(masked, axis=1, keepdims=True)
    idx1 = jnp.min(jnp.where(masked == v1, lane, E), axis=1, keepdims=True)
    sel1 = (lane == idx1)
    masked2 = jnp.where(sel1, NEG, masked)
    v2 = jnp.max(masked2, axis=1, keepdims=True)
    idx2 = jnp.min(jnp.where(masked2 == v2, lane, E), axis=1, keepdims=True)
    sel2 = (lane == idx2)

    w1v = jnp.sum(jnp.where(sel1, scores, 0.0), axis=1, keepdims=True)
    w2v = jnp.sum(jnp.where(sel2, scores, 0.0), axis=1, keepdims=True)
    denom = w1v + w2v + 1e-20
    return sel1, sel2, ROUTED_SCALING * w1v / denom, ROUTED_SCALING * w2v / denom


def _routing_kernel(x_ref, gw_ref, gb_ref, eidx_ref, rnk_ref, wts_ref,
                    cnt_ref, run_ref):
    t = pl.program_id(0)
    bT = x_ref.shape[0]
    E = gw_ref.shape[0]

    @pl.when(t == 0)
    def _():
        run_ref[...] = jnp.zeros_like(run_ref)

    sel1, sel2, w1n, w2n = _gate_topk(x_ref[...], gw_ref[...], gb_ref[...])
    lane = lax.broadcasted_iota(jnp.int32, (bT, E), 1)
    idx1 = jnp.sum(jnp.where(sel1, lane, 0), axis=1, keepdims=True)
    idx2 = jnp.sum(jnp.where(sel2, lane, 0), axis=1, keepdims=True)

    oh = sel1.astype(jnp.float32) + sel2.astype(jnp.float32)
    row = lax.broadcasted_iota(jnp.int32, (bT, bT), 0)
    col = lax.broadcasted_iota(jnp.int32, (bT, bT), 1)
    ltri = jnp.where(row > col, 1.0, 0.0)
    prefix = lax.dot_general(
        ltri, oh, (((1,), (0,)), ((), ())),
        preferred_element_type=jnp.float32)
    base = run_ref[...] + prefix
    r1 = jnp.sum(jnp.where(sel1, base, 0.0), axis=1, keepdims=True)
    r2 = jnp.sum(jnp.where(sel2, base, 0.0), axis=1, keepdims=True)
    run_new = run_ref[...] + jnp.sum(oh, axis=0, keepdims=True)
    run_ref[...] = run_new

    eidx_ref[...] = jnp.concatenate([idx1, idx2], axis=1)
    rnk_ref[...] = jnp.concatenate([r1, r2], axis=1).astype(jnp.int32)
    wts_ref[...] = jnp.concatenate([w1n, w2n], axis=1)
    cnt_ref[...] = run_new.astype(jnp.int32)


def _grouped_mlp_kernel(be_ref, valid_ref, xs_ref, w1_ref, w2_ref, ys_ref):
    b = pl.program_id(0)

    @pl.when(valid_ref[b] == 1)
    def _():
        h = _relu2(lax.dot_general(
            xs_ref[...].astype(jnp.bfloat16),
            w1_ref[0].astype(jnp.bfloat16), (((1,), (1,)), ((), ())),
            preferred_element_type=jnp.float32))
        ys_ref[...] = lax.dot_general(
            h.astype(jnp.bfloat16),
            w2_ref[0].astype(jnp.bfloat16), (((1,), (1,)), ((), ())),
            preferred_element_type=jnp.float32)


def _shared_mlp_kernel(x_ref, sw1_ref, sw2_ref, out_ref):
    s = _relu2(lax.dot_general(
        x_ref[...], sw1_ref[...], (((1,), (1,)), ((), ())),
        preferred_element_type=jnp.float32))
    out_ref[...] = lax.dot_general(
        s, sw2_ref[...], (((1,), (1,)), ((), ())),
        preferred_element_type=jnp.float32)


def _combine_kernel(sh_ref, yg_ref, wts_ref, out_ref):
    y0 = yg_ref[:, 0, :]
    y1 = yg_ref[:, 1, :]
    w0 = wts_ref[:, 0:1]
    w1 = wts_ref[:, 1:2]
    out_ref[...] = sh_ref[...] + w0 * y0 + w1 * y1


def _sc_dispatch(x, tok, pos, S, chunk=64):
    T, D = x.shape
    B = tok.shape[0]
    info = plsc.get_sparse_core_info()
    NC, NS = info.num_cores, info.num_subcores
    NW = NC * NS
    assert B % (8 * NW) == 0 and D % 16 == 0
    b_per_w = B // NW
    assert b_per_w % chunk == 0 and chunk <= 128 and chunk % 8 == 0
    n_chunks = b_per_w // chunk
    mesh = plsc.VectorSubcoreMesh(core_axis_name="c", subcore_axis_name="s")

    @functools.partial(
        pl.kernel, mesh=mesh,
        out_type=jax.ShapeDtypeStruct((S, D), jnp.float32),
        scratch_types=[
            pltpu.VMEM((chunk,), jnp.int32),
            pltpu.VMEM((chunk,), jnp.int32),
            pltpu.VMEM((chunk, D), jnp.float32),
            pltpu.SemaphoreType.DMA,
        ],
    )
    def dispatch_k(x_hbm, tok_hbm, pos_hbm, xs_hbm, tidx_v, pidx_v, rows_v,
                   sem):
        wid = lax.axis_index("s") * NC + lax.axis_index("c")
        base = wid * b_per_w
        for c in range(n_chunks):
            cbase = base + c * chunk
            pltpu.sync_copy(tok_hbm.at[pl.ds(cbase, chunk)], tidx_v)
            pltpu.sync_copy(pos_hbm.at[pl.ds(cbase, chunk)], pidx_v)
            pltpu.async_copy(x_hbm.at[tidx_v], rows_v, sem).wait()
            pltpu.async_copy(rows_v, xs_hbm.at[pidx_v], sem).wait()

    return dispatch_k(x, tok, pos)


def _sc_gather(table, idx, chunk=64):
    V, D = table.shape
    B = idx.shape[0]
    info = plsc.get_sparse_core_info()
    NC, NS = info.num_cores, info.num_subcores
    NW = NC * NS
    assert B % (8 * NW) == 0 and D % 16 == 0
    b_per_w = B // NW
    assert b_per_w % chunk == 0 and chunk <= 128 and chunk % 8 == 0
    n_chunks = b_per_w // chunk
    mesh = plsc.VectorSubcoreMesh(core_axis_name="c", subcore_axis_name="s")

    @functools.partial(
        pl.kernel, mesh=mesh,
        out_type=jax.ShapeDtypeStruct((B, D), jnp.float32),
        scratch_types=[
            pltpu.VMEM((chunk,), jnp.int32),
            pltpu.VMEM((chunk, D), jnp.float32),
            pltpu.SemaphoreType.DMA,
        ],
    )
    def gather_k(table_hbm, idx_hbm, out_hbm, idx_v, rows_v, sem):
        wid = lax.axis_index("s") * NC + lax.axis_index("c")
        base = wid * b_per_w
        for c in range(n_chunks):
            cbase = base + c * chunk
            pltpu.sync_copy(idx_hbm.at[pl.ds(cbase, chunk)], idx_v)
            pltpu.async_copy(table_hbm.at[idx_v], rows_v, sem).wait()
            pltpu.sync_copy(rows_v, out_hbm.at[pl.ds(cbase, chunk)])

    return gather_k(table, idx)


@jax.jit
def kernel(hidden_states, gate_w, gate_bias, w1, w2, shared_w1, shared_w2):
    x = hidden_states
    T, D = x.shape
    E, I, _ = w1.shape
    SI = shared_w1.shape[0]
    NB = (T * TOP_K) // BLK + E
    S = NB * BLK

    bT = min(512, T)
    eidx, rnk, wts, counts = pl.pallas_call(
        _routing_kernel,
        grid=(T // bT,),
        in_specs=[
            pl.BlockSpec((bT, D), lambda t: (t, 0)),
            pl.BlockSpec((E, D), lambda t: (0, 0)),
            pl.BlockSpec((1, E), lambda t: (0, 0)),
        ],
        out_specs=[
            pl.BlockSpec((bT, TOP_K), lambda t: (t, 0)),
            pl.BlockSpec((bT, TOP_K), lambda t: (t, 0)),
            pl.BlockSpec((bT, TOP_K), lambda t: (t, 0)),
            pl.BlockSpec((1, E), lambda t: (0, 0)),
        ],
        out_shape=[
            jax.ShapeDtypeStruct((T, TOP_K), jnp.int32),
            jax.ShapeDtypeStruct((T, TOP_K), jnp.int32),
            jax.ShapeDtypeStruct((T, TOP_K), jnp.float32),
            jax.ShapeDtypeStruct((1, E), jnp.int32),
        ],
        scratch_shapes=[pltpu.VMEM((1, E), jnp.float32)],
    )(x, gate_w, gate_bias.reshape(1, E))

    c = counts[0]
    nb = (c + BLK - 1) // BLK
    total_blocks = jnp.sum(nb)
    block_expert = jnp.repeat(
        jnp.arange(E, dtype=jnp.int32), nb, total_repeat_length=NB)
    block_valid = (jnp.arange(NB, dtype=jnp.int32)
                   < total_blocks).astype(jnp.int32)
    offsets = (jnp.cumsum(nb) - nb).astype(jnp.int32) * BLK
    pos = offsets[eidx] + rnk
    pos_flat = pos.reshape(-1)
    tok_flat = (jnp.arange(T * TOP_K, dtype=jnp.int32) // TOP_K)

    bT2 = min(512, T)
    shared_y = pl.pallas_call(
        _shared_mlp_kernel,
        grid=(T // bT2,),
        in_specs=[
            pl.BlockSpec((bT2, D), lambda t: (t, 0)),
            pl.BlockSpec((SI, D), lambda t: (0, 0)),
            pl.BlockSpec((D, SI), lambda t: (0, 0)),
        ],
        out_specs=pl.BlockSpec((bT2, D), lambda t: (t, 0)),
        out_shape=jax.ShapeDtypeStruct((T, D), jnp.float32),
    )(x, shared_w1, shared_w2)

    xs = _sc_dispatch(x, tok_flat, pos_flat, S)

    ys = pl.pallas_call(
        _grouped_mlp_kernel,
        grid_spec=pltpu.PrefetchScalarGridSpec(
            num_scalar_prefetch=2,
            grid=(NB,),
            in_specs=[
                pl.BlockSpec((BLK, D), lambda b, be, bv: (b, 0)),
                pl.BlockSpec((1, I, D), lambda b, be, bv: (be[b], 0, 0)),
                pl.BlockSpec((1, D, I), lambda b, be, bv: (be[b], 0, 0)),
            ],
            out_specs=pl.BlockSpec((BLK, D), lambda b, be, bv: (b, 0)),
        ),
        out_shape=jax.ShapeDtypeStruct((S, D), jnp.float32),
    )(block_expert, block_valid, xs, w1, w2)

    yg = _sc_gather(ys, pos_flat)
    yg = yg.reshape(T, TOP_K, D)

    out = pl.pallas_call(
        _combine_kernel,
        grid=(T // bT2,),
        in_specs=[
            pl.BlockSpec((bT2, D), lambda t: (t, 0)),
            pl.BlockSpec((bT2, TOP_K, D), lambda t: (t, 0, 0)),
            pl.BlockSpec((bT2, TOP_K), lambda t: (t, 0)),
        ],
        out_specs=pl.BlockSpec((bT2, D), lambda t: (t, 0)),
        out_shape=jax.ShapeDtypeStruct((T, D), jnp.float32),
    )(shared_y, yg, wts)
    return out
